# Initial kernel scaffold; baseline (speedup 1.0000x reference)
import jax
import jax.numpy as jnp
from jax.experimental import pallas as pl


def kernel(x, W, b):
    raise NotImplementedError("write your pallas kernel here")


# speedup vs baseline: 3.9997x; 3.9997x over previous
import jax
import jax.numpy as jnp
from jax.experimental import pallas as pl
from jax.experimental.pallas import tpu as pltpu

_CAT = (0, 5, 10, 15)
_K = 101
_B, _T, _F = 1024, 20, 32
_BS = 128
_G = _B // _BS


def _kern(xsel_ref, o0, o1, o2, o3, mask_ref):
    step = pl.program_id(0)

    @pl.when(step == 0)
    def _compute_masks():
        lane = jax.lax.broadcasted_iota(jnp.int32, (_B, 128), 1)
        for f in range(4):
            v = xsel_ref[:, f * _T:(f + 1) * _T]
            i = v.astype(jnp.int32) + 1
            i = jnp.where(i < 0, i + _K, i)
            acc = jnp.zeros((_B, 128), jnp.float32)
            for t in range(_T):
                col = i[:, t:t + 1]
                acc = jnp.maximum(acc, (col == lane).astype(jnp.float32))
            mask_ref[f:f + 1, :] = jnp.max(acc, axis=0, keepdims=True)

    for f, o in enumerate((o0, o1, o2, o3)):
        m = mask_ref[f:f + 1, 0:_K]
        o[...] = jnp.broadcast_to(m.reshape(1, 1, _K), (_BS, _T, _K))


def kernel(x, W, b):
    xsel = jnp.concatenate([x[:, :, c] for c in _CAT], axis=1)
    out_shape = [jax.ShapeDtypeStruct((_B, _T, _K), jnp.float32)] * 4
    c = pl.pallas_call(
        _kern,
        grid=(_G,),
        in_specs=[pl.BlockSpec((_B, 4 * _T), lambda i: (0, 0))],
        out_specs=[pl.BlockSpec((_BS, _T, _K), lambda i: (i, 0, 0))] * 4,
        out_shape=out_shape,
        scratch_shapes=[pltpu.VMEM((8, 128), jnp.float32)],
    )(xsel)
    return (x, x, c[0], c[1], c[2], c[3])
